# Initial kernel scaffold; baseline (speedup 1.0000x reference)
import jax
import jax.numpy as jnp
from jax.experimental import pallas as pl


def kernel(positive_sample, negative_sample, mode, entity_embedding, relation_embedding):
    raise NotImplementedError("write your pallas kernel here")


# speedup vs baseline: 2.2679x; 2.2679x over previous
import functools

import jax
import jax.numpy as jnp
from jax import lax
from jax.experimental import pallas as pl
from jax.experimental.pallas import tpu as pltpu
from jax.experimental.pallas import tpu_sc as plsc

_GAMMA = 12.0
_LANES = 16


def _make_score_kernel(batch, hidden):
    info = plsc.get_sparse_core_info()
    nc, ns = info.num_cores, info.num_subcores
    nw = nc * ns
    assert batch % nw == 0 and hidden % _LANES == 0
    bpw = batch // nw

    @functools.partial(
        pl.kernel,
        mesh=plsc.VectorSubcoreMesh(core_axis_name="c", subcore_axis_name="s"),
        out_type=jax.ShapeDtypeStruct((batch,), jnp.float32),
        compiler_params=pltpu.CompilerParams(needs_layout_passes=False),
        scratch_types=[
            pltpu.VMEM((bpw,), jnp.int32),
            pltpu.VMEM((bpw,), jnp.int32),
            pltpu.VMEM((bpw,), jnp.int32),
            pltpu.VMEM((bpw, hidden), jnp.float32),
            pltpu.VMEM((bpw, hidden), jnp.float32),
            pltpu.VMEM((bpw, hidden), jnp.float32),
            pltpu.VMEM((bpw, _LANES), jnp.float32),
            pltpu.VMEM((bpw,), jnp.float32),
            pltpu.SemaphoreType.DMA,
        ],
    )
    def score(h_idx_hbm, r_idx_hbm, t_idx_hbm, ent_hbm, rel_hbm, out_hbm,
              h_idx_v, r_idx_v, t_idx_v, h_v, r_v, t_v, acc_v, out_v, sem):
        wid = lax.axis_index("s") * nc + lax.axis_index("c")
        base = wid * bpw
        pltpu.sync_copy(h_idx_hbm.at[pl.ds(base, bpw)], h_idx_v)
        pltpu.sync_copy(r_idx_hbm.at[pl.ds(base, bpw)], r_idx_v)
        pltpu.sync_copy(t_idx_hbm.at[pl.ds(base, bpw)], t_idx_v)
        ch = pltpu.async_copy(ent_hbm.at[h_idx_v], h_v, sem)
        cr = pltpu.async_copy(rel_hbm.at[r_idx_v], r_v, sem)
        ct = pltpu.async_copy(ent_hbm.at[t_idx_v], t_v, sem)
        ch.wait()
        cr.wait()
        ct.wait()
        for i in range(bpw):
            acc = jnp.zeros((_LANES,), jnp.float32)
            for d in range(hidden // _LANES):
                sl = pl.ds(d * _LANES, _LANES)
                acc = acc + jnp.abs(h_v[i, sl] + (r_v[i, sl] - t_v[i, sl]))
            acc_v[i] = acc
        lane_ids = lax.iota(jnp.int32, _LANES)
        for g in range(bpw // _LANES):
            rows = lane_ids + g * _LANES
            tot = jnp.zeros((_LANES,), jnp.float32)
            for j in range(_LANES):
                cols = jnp.full((_LANES,), j, jnp.int32)
                tot = tot + plsc.load_gather(acc_v, [rows, cols])
            out_v[pl.ds(g * _LANES, _LANES)] = _GAMMA - tot
        pltpu.sync_copy(out_v, out_hbm.at[pl.ds(base, bpw)])

    return score


def kernel(positive_sample, negative_sample, mode, entity_embedding, relation_embedding):
    del negative_sample, mode
    batch = positive_sample.shape[0]
    hidden = entity_embedding.shape[1]
    h_idx = positive_sample[:, 0]
    r_idx = positive_sample[:, 1]
    t_idx = positive_sample[:, 2]
    score = _make_score_kernel(batch, hidden)
    out = score(h_idx, r_idx, t_idx, entity_embedding, relation_embedding)
    return out.reshape(batch, 1)


# speedup vs baseline: 2.2974x; 1.0130x over previous
import functools

import jax
import jax.numpy as jnp
from jax import lax
from jax.experimental import pallas as pl
from jax.experimental.pallas import tpu as pltpu
from jax.experimental.pallas import tpu_sc as plsc

_GAMMA = 12.0
_LANES = 16


def _make_score_kernel(batch, hidden):
    info = plsc.get_sparse_core_info()
    nc, ns = info.num_cores, info.num_subcores
    nw = nc * ns
    assert batch % (nw * _LANES) == 0 and hidden % _LANES == 0
    bpw = batch // nw
    ngrp = bpw // _LANES

    @functools.partial(
        pl.kernel,
        mesh=plsc.VectorSubcoreMesh(core_axis_name="c", subcore_axis_name="s"),
        out_type=jax.ShapeDtypeStruct((batch,), jnp.float32),
        compiler_params=pltpu.CompilerParams(needs_layout_passes=False),
        scratch_types=[
            pltpu.VMEM((bpw, 3), jnp.int32),
            pltpu.VMEM((2 * bpw,), jnp.int32),
            pltpu.VMEM((bpw,), jnp.int32),
            pltpu.VMEM((2 * bpw, hidden), jnp.float32),
            pltpu.VMEM((bpw, hidden), jnp.float32),
            pltpu.VMEM((bpw, _LANES), jnp.float32),
            pltpu.VMEM((bpw,), jnp.float32),
            pltpu.SemaphoreType.DMA,
        ],
    )
    def score(ps_hbm, ent_hbm, rel_hbm, out_hbm,
              ps_v, ht_idx_v, r_idx_v, ht_v, r_v, acc_v, out_v, sem):
        wid = lax.axis_index("s") * nc + lax.axis_index("c")
        base = wid * bpw
        pltpu.async_copy(ps_hbm.at[pl.ds(base, bpw)], ps_v, sem).wait()
        lane_ids = lax.iota(jnp.int32, _LANES)
        for g in range(ngrp):
            rows = lane_ids + g * _LANES
            sl = pl.ds(g * _LANES, _LANES)
            ht_idx_v[sl] = plsc.load_gather(ps_v, [rows, jnp.zeros((_LANES,), jnp.int32)])
            ht_idx_v[pl.ds(bpw + g * _LANES, _LANES)] = plsc.load_gather(
                ps_v, [rows, jnp.full((_LANES,), 2, jnp.int32)])
            r_idx_v[sl] = plsc.load_gather(ps_v, [rows, jnp.ones((_LANES,), jnp.int32)])
        ce = pltpu.async_copy(ent_hbm.at[ht_idx_v], ht_v, sem)
        cr = pltpu.async_copy(rel_hbm.at[r_idx_v], r_v, sem)
        ce.wait()
        cr.wait()
        for i in range(bpw):
            acc = jnp.zeros((_LANES,), jnp.float32)
            for d in range(hidden // _LANES):
                sl = pl.ds(d * _LANES, _LANES)
                acc = acc + jnp.abs(ht_v[i, sl] + (r_v[i, sl] - ht_v[bpw + i, sl]))
            acc_v[i] = acc
        for g in range(ngrp):
            rows = lane_ids + g * _LANES
            tot = jnp.zeros((_LANES,), jnp.float32)
            for j in range(_LANES):
                cols = jnp.full((_LANES,), j, jnp.int32)
                tot = tot + plsc.load_gather(acc_v, [rows, cols])
            out_v[pl.ds(g * _LANES, _LANES)] = _GAMMA - tot
        pltpu.sync_copy(out_v, out_hbm.at[pl.ds(base, bpw)])

    return score


def kernel(positive_sample, negative_sample, mode, entity_embedding, relation_embedding):
    del negative_sample, mode
    batch = positive_sample.shape[0]
    hidden = entity_embedding.shape[1]
    score = _make_score_kernel(batch, hidden)
    out = score(positive_sample, entity_embedding, relation_embedding)
    return out.reshape(batch, 1)


# speedup vs baseline: 2.6028x; 1.1329x over previous
import functools

import jax
import jax.numpy as jnp
from jax import lax
from jax.experimental import pallas as pl
from jax.experimental.pallas import tpu as pltpu
from jax.experimental.pallas import tpu_sc as plsc

_GAMMA = 12.0
_LANES = 16


def _make_score_kernel(batch, hidden):
    info = plsc.get_sparse_core_info()
    nc, ns = info.num_cores, info.num_subcores
    nw = nc * ns
    assert batch % (nw * _LANES) == 0 and hidden % _LANES == 0
    bpw = batch // nw
    ngrp = bpw // _LANES

    @functools.partial(
        pl.kernel,
        mesh=plsc.VectorSubcoreMesh(core_axis_name="c", subcore_axis_name="s"),
        out_type=jax.ShapeDtypeStruct((batch,), jnp.float32),
        compiler_params=pltpu.CompilerParams(needs_layout_passes=False),
        scratch_types=[
            pltpu.VMEM((bpw, 3), jnp.int32),
            pltpu.VMEM((2 * bpw,), jnp.int32),
            pltpu.VMEM((bpw,), jnp.int32),
            pltpu.VMEM((2 * bpw, hidden), jnp.float32),
            pltpu.VMEM((bpw, hidden), jnp.float32),
            pltpu.VMEM((bpw, _LANES), jnp.float32),
            pltpu.VMEM((bpw,), jnp.float32),
            pltpu.SemaphoreType.DMA,
        ],
    )
    def score(ps_hbm, ent_hbm, rel_hbm, out_hbm,
              ps_v, ht_idx_v, r_idx_v, ht_v, r_v, acc_v, out_v, sem):
        wid = lax.axis_index("s") * nc + lax.axis_index("c")
        base = wid * bpw
        pltpu.async_copy(ps_hbm.at[pl.ds(base, bpw)], ps_v, sem).wait()
        lane_ids = lax.iota(jnp.int32, _LANES)
        for g in range(ngrp):
            rows = lane_ids + g * _LANES
            sl = pl.ds(g * _LANES, _LANES)
            ht_idx_v[sl] = plsc.load_gather(ps_v, [rows, jnp.zeros((_LANES,), jnp.int32)])
            ht_idx_v[pl.ds(bpw + g * _LANES, _LANES)] = plsc.load_gather(
                ps_v, [rows, jnp.full((_LANES,), 2, jnp.int32)])
            r_idx_v[sl] = plsc.load_gather(ps_v, [rows, jnp.ones((_LANES,), jnp.int32)])
        ce = pltpu.async_copy(ent_hbm.at[ht_idx_v], ht_v, sem)
        cr = pltpu.async_copy(rel_hbm.at[r_idx_v], r_v, sem)
        ce.wait()
        cr.wait()
        @plsc.parallel_loop(0, bpw, 1, unroll=2)
        def _stage1(i):
            acc = jnp.zeros((_LANES,), jnp.float32)
            for d in range(hidden // _LANES):
                sl = pl.ds(d * _LANES, _LANES)
                acc = acc + jnp.abs(ht_v[i, sl] + (r_v[i, sl] - ht_v[bpw + i, sl]))
            acc_v[i] = acc
        for g in range(ngrp):
            rows = lane_ids + g * _LANES
            tot = jnp.zeros((_LANES,), jnp.float32)
            for j in range(_LANES):
                cols = jnp.full((_LANES,), j, jnp.int32)
                tot = tot + plsc.load_gather(acc_v, [rows, cols])
            out_v[pl.ds(g * _LANES, _LANES)] = _GAMMA - tot
        pltpu.sync_copy(out_v, out_hbm.at[pl.ds(base, bpw)])

    return score


def kernel(positive_sample, negative_sample, mode, entity_embedding, relation_embedding):
    del negative_sample, mode
    batch = positive_sample.shape[0]
    hidden = entity_embedding.shape[1]
    score = _make_score_kernel(batch, hidden)
    out = score(positive_sample, entity_embedding, relation_embedding)
    return out.reshape(batch, 1)
